# Initial kernel scaffold; baseline (speedup 1.0000x reference)
import jax
import jax.numpy as jnp
from jax.experimental import pallas as pl


def kernel(x, embedding, new_embedding):
    raise NotImplementedError("write your pallas kernel here")


# speedup vs baseline: 3.0053x; 3.0053x over previous
import functools

import jax
import jax.numpy as jnp
from jax import lax
from jax.experimental import pallas as pl
from jax.experimental.pallas import tpu as pltpu
from jax.experimental.pallas import tpu_sc as plsc

_NUM_EMB = 100000
_NUM_NEW = 1024
_FEATURES = 128
_BATCH = 16384
_HIST = 50

_NC, _NS = 2, 16
_NW = _NC * _NS
_B = _BATCH * _HIST
_IDXW = 128
_ROWS_PER_W = _B // _NW
_GROUPS = _ROWS_PER_W // _IDXW


def _gather_body(x_hbm, table_hbm, out_hbm, idx_v, buf, gsem):
    c = lax.axis_index("c")
    s = lax.axis_index("s")
    wid = s * _NC + c
    irow0 = wid * _GROUPS
    out_base = wid * _ROWS_PER_W

    pltpu.sync_copy(x_hbm.at[pl.ds(irow0, _GROUPS)], idx_v)

    def body(i, _):
        pltpu.async_copy(table_hbm.at[idx_v.at[i]], buf, gsem).wait()
        pltpu.sync_copy(buf, out_hbm.at[pl.ds(out_base + i * _IDXW, _IDXW)])
        return 0

    lax.fori_loop(0, _GROUPS, body, 0)


def kernel(x, embedding, new_embedding):
    table = jnp.concatenate([embedding, new_embedding], axis=0)
    x2d = x.reshape(-1).astype(jnp.int32).reshape(_B // _IDXW, _IDXW)

    mesh = plsc.VectorSubcoreMesh(core_axis_name="c", subcore_axis_name="s")
    run = pl.kernel(
        _gather_body,
        out_type=jax.ShapeDtypeStruct((_B, _FEATURES), jnp.float32),
        mesh=mesh,
        scratch_types=[
            pltpu.VMEM((_GROUPS, _IDXW), jnp.int32),
            pltpu.VMEM((_IDXW, _FEATURES), jnp.float32),
            pltpu.SemaphoreType.DMA,
        ],
    )
    out = run(x2d, table)
    return out.reshape(_BATCH, _HIST, _FEATURES)


# speedup vs baseline: 3.3882x; 1.1274x over previous
import functools

import jax
import jax.numpy as jnp
from jax import lax
from jax.experimental import pallas as pl
from jax.experimental.pallas import tpu as pltpu
from jax.experimental.pallas import tpu_sc as plsc

_NUM_EMB = 100000
_NUM_NEW = 1024
_FEATURES = 128
_BATCH = 16384
_HIST = 50

_NC, _NS = 2, 16
_NW = _NC * _NS
_B = _BATCH * _HIST
_IDXW = 128
_ROWS_PER_W = _B // _NW
_GROUPS = _ROWS_PER_W // _IDXW


_NBUF = 4
_GG = _GROUPS // _NBUF


def _gather_body(x_hbm, table_hbm, out_hbm, idx_v, bufs, gsems, osems):
    c = lax.axis_index("c")
    s = lax.axis_index("s")
    wid = s * _NC + c
    irow0 = wid * _GROUPS
    out_base = wid * _ROWS_PER_W

    pltpu.sync_copy(x_hbm.at[pl.ds(irow0, _GROUPS)], idx_v)

    def wait_gather(b):
        pltpu.make_async_copy(
            out_hbm.at[pl.ds(out_base, _IDXW)], bufs[b], gsems[b]).wait()

    def wait_scatter(b):
        pltpu.make_async_copy(
            bufs[b], out_hbm.at[pl.ds(out_base, _IDXW)], osems[b]).wait()

    for b in range(_NBUF):
        pltpu.async_copy(table_hbm.at[idx_v.at[b]], bufs[b], gsems[b])

    def body(gg, _):
        for b in range(_NBUF):
            i = gg * _NBUF + b
            wait_gather(b)
            pltpu.async_copy(
                bufs[b], out_hbm.at[pl.ds(out_base + i * _IDXW, _IDXW)],
                osems[b])

            @pl.when(gg < _GG - 1)
            def _():
                wait_scatter(b)
                pltpu.async_copy(
                    table_hbm.at[idx_v.at[i + _NBUF]], bufs[b], gsems[b])
        return 0

    lax.fori_loop(0, _GG, body, 0)
    for b in range(_NBUF):
        wait_scatter(b)


def kernel(x, embedding, new_embedding):
    table = jnp.concatenate([embedding, new_embedding], axis=0)
    x2d = x.reshape(-1).astype(jnp.int32).reshape(_B // _IDXW, _IDXW)

    mesh = plsc.VectorSubcoreMesh(core_axis_name="c", subcore_axis_name="s")
    run = pl.kernel(
        _gather_body,
        out_type=jax.ShapeDtypeStruct((_B, _FEATURES), jnp.float32),
        mesh=mesh,
        scratch_types=[
            pltpu.VMEM((_GROUPS, _IDXW), jnp.int32),
            tuple(pltpu.VMEM((_IDXW, _FEATURES), jnp.float32)
                  for _ in range(_NBUF)),
            tuple(pltpu.SemaphoreType.DMA for _ in range(_NBUF)),
            tuple(pltpu.SemaphoreType.DMA for _ in range(_NBUF)),
        ],
    )
    out = run(x2d, table)
    return out.reshape(_BATCH, _HIST, _FEATURES)
